# baseline (speedup 1.0000x reference)
import functools

import jax
import jax.numpy as jnp
from jax import lax
from jax.experimental import pallas as pl
from jax.experimental.pallas import tpu as pltpu
from jax.experimental.pallas import tpu_sc as plsc


@functools.cache
def _make_gather(V, D, B):
    info = plsc.get_sparse_core_info()
    NC, NS = info.num_cores, info.num_subcores
    NW = NC * NS
    assert B % NW == 0 and (B // NW) % 8 == 0
    b_per_w = B // NW
    mesh = plsc.VectorSubcoreMesh(core_axis_name="c", subcore_axis_name="s")

    @functools.partial(
        pl.kernel,
        mesh=mesh,
        out_type=jax.ShapeDtypeStruct((B, D), jnp.float32),
        scratch_types=[
            pltpu.VMEM((b_per_w,), jnp.int32),
            pltpu.VMEM((b_per_w, D), jnp.float32),
            pltpu.SemaphoreType.DMA,
        ],
        compiler_params=pltpu.CompilerParams(use_tc_tiling_on_sc=False),
    )
    def gather_kernel(idx_hbm, table_hbm, out_hbm, idx_v, rows_v, sem):
        wid = lax.axis_index("s") * NC + lax.axis_index("c")
        base = wid * b_per_w
        pltpu.sync_copy(idx_hbm.at[pl.ds(base, b_per_w)], idx_v)
        pltpu.async_copy(table_hbm.at[idx_v], rows_v, sem).wait()
        pltpu.sync_copy(rows_v, out_hbm.at[pl.ds(base, b_per_w)])

    return gather_kernel


@jax.jit
def kernel(g, table):
    f = _make_gather(table.shape[0], table.shape[1], g.shape[0])
    return f(g.astype(jnp.int32), table)


# speedup vs baseline: 1.0339x; 1.0339x over previous
import functools

import jax
import jax.numpy as jnp
from jax import lax
from jax.experimental import pallas as pl
from jax.experimental.pallas import tpu as pltpu
from jax.experimental.pallas import tpu_sc as plsc

_LANES = 16


@functools.cache
def _make_gather(V, D, B):
    info = plsc.get_sparse_core_info()
    NC, NS = info.num_cores, info.num_subcores
    NW = NC * NS
    assert B % (_LANES * NW) == 0
    b_per_w = B // NW
    n_grp = b_per_w // _LANES
    mesh = plsc.VectorSubcoreMesh(core_axis_name="c", subcore_axis_name="s")

    @functools.partial(
        pl.kernel,
        mesh=mesh,
        out_type=jax.ShapeDtypeStruct((B, D), jnp.float32),
        scratch_types=[
            pltpu.VMEM((b_per_w,), jnp.int32),
            pltpu.SemaphoreType.DMA,
        ],
        compiler_params=pltpu.CompilerParams(needs_layout_passes=False),
    )
    def gather_kernel(idx_hbm, table_hbm, out_hbm, g_v, sem):
        wid = lax.axis_index("s") * NC + lax.axis_index("c")
        base = wid * b_per_w
        pltpu.sync_copy(idx_hbm.at[pl.ds(base, b_per_w)], g_v)
        lane = lax.iota(jnp.int32, _LANES)

        def fire(j, _):
            g16 = g_v[pl.ds(j * _LANES, _LANES)]
            row0 = base + j * _LANES
            for l in range(_LANES):
                g = jnp.sum(jnp.where(lane == l, g16, 0))
                pltpu.async_copy(table_hbm.at[g], out_hbm.at[row0 + l], sem)
            return 0

        lax.fori_loop(0, n_grp, fire, 0)

        def drain(i, _):
            pltpu.make_async_copy(table_hbm.at[0], out_hbm.at[base + i], sem).wait()
            return 0

        lax.fori_loop(0, b_per_w, drain, 0)

    return gather_kernel


@jax.jit
def kernel(g, table):
    f = _make_gather(table.shape[0], table.shape[1], g.shape[0])
    return f(g.astype(jnp.int32), table)


# speedup vs baseline: 1.7263x; 1.6697x over previous
import functools

import jax
import jax.numpy as jnp
from jax import lax
from jax.experimental import pallas as pl
from jax.experimental.pallas import tpu as pltpu
from jax.experimental.pallas import tpu_sc as plsc

_LANES = 16


@functools.cache
def _make_gather(V, D, B):
    info = plsc.get_sparse_core_info()
    NC, NS = info.num_cores, info.num_subcores
    NW = NC * NS
    assert B % (_LANES * NW) == 0 and B % (8 * NW) == 0
    b_per_w = B // NW
    mesh = plsc.VectorSubcoreMesh(core_axis_name="c", subcore_axis_name="s")

    @functools.partial(
        pl.kernel,
        mesh=mesh,
        out_type=jax.ShapeDtypeStruct((B // 8, 8, D), jnp.float32),
        scratch_types=[
            pltpu.VMEM((b_per_w,), jnp.int32),
            pltpu.VMEM((b_per_w // 8, 8, D), jnp.float32),
            pltpu.SemaphoreType.DMA,
        ],
        compiler_params=pltpu.CompilerParams(needs_layout_passes=False),
    )
    def gather_kernel(idx_hbm, table_hbm, out_hbm, g_v, rows_v, sem):
        wid = lax.axis_index("s") * NC + lax.axis_index("c")
        base = wid * b_per_w
        pltpu.sync_copy(idx_hbm.at[pl.ds(base, b_per_w)], g_v)

        def fire(j, _):
            g16 = g_v[pl.ds(j * _LANES, _LANES)]
            i0 = j * _LANES
            for l in range(_LANES):
                i = i0 + l
                pltpu.async_copy(table_hbm.at[g16[l]], rows_v.at[i // 8, i % 8], sem)
            return 0

        lax.fori_loop(0, b_per_w // _LANES, fire, 0)

        def drain(i, _):
            pltpu.make_async_copy(table_hbm.at[0], rows_v.at[0, 0], sem).wait()
            return 0

        lax.fori_loop(0, b_per_w, drain, 0)
        pltpu.sync_copy(rows_v, out_hbm.at[pl.ds(wid * (b_per_w // 8), b_per_w // 8)])

    return gather_kernel


@jax.jit
def kernel(g, table):
    V, D = table.shape
    B = g.shape[0]
    f = _make_gather(V, D, B)
    return f(g.astype(jnp.int32), table).reshape(B, D)
